# baseline (speedup 1.0000x reference)
import functools

import jax
import jax.numpy as jnp
from jax.experimental import pallas as pl
from jax.experimental.pallas import tpu as pltpu

_EPS = 1e-5
_NEG = -1e30


def _affine_from_stats(st, g, b, cin, count):
    s = st[0, :cin]
    ss = st[1, :cin]
    mean = s / count
    var = ss / count - mean * mean
    a = g * jax.lax.rsqrt(var + _EPS)
    sh = b - mean * a
    return a, sh


def _pack_stats(acc_s, acc_ss, cout):
    def row(v):
        v = v.reshape(1, -1)
        if cout < 128:
            v = jnp.concatenate(
                [v, jnp.zeros((1, 128 - cout), jnp.float32)], axis=1)
        return v
    return jnp.concatenate(
        [row(acc_s), row(acc_ss), jnp.zeros((6, 128), jnp.float32)], axis=0)


def _pool_border(y, TH, W, cout):
    yp = jnp.max(y.reshape(TH // 2, 2, W // 2, 2, cout), axis=(1, 3))
    neg = jnp.full((TH // 2, 1, cout), _NEG, jnp.float32)
    return jnp.concatenate([neg, yp, neg], axis=1)


def _stage1_kernel(xs_ref, w_ref, cb_ref, out_ref, st_ref):
    @pl.when((pl.program_id(0) == 0) & (pl.program_id(1) == 0))
    def _():
        st_ref[...] = jnp.zeros_like(st_ref)

    xs = xs_ref[0, 0]
    taps = [xs[dy:dy + 28, dx:dx + 224, :]
            for dy in range(3) for dx in range(3)]
    patches = jnp.concatenate(taps, axis=-1).reshape(28 * 224, 36)
    y = jnp.dot(patches, w_ref[...], preferred_element_type=jnp.float32)
    y = y + cb_ref[...]
    st_ref[...] = st_ref[...] + _pack_stats(
        jnp.sum(y, axis=0), jnp.sum(y * y, axis=0), 32)
    out_ref[0] = _pool_border(y, 28, 224, 32)


def _stageN_kernel(x_ref, st_in_ref, g_ref, b_ref, w_ref, cb_ref,
                   out_ref, st_ref, *, H, W, Cin, Cout, strips, count):
    @pl.when(pl.program_id(0) == 0)
    def _():
        st_ref[...] = jnp.zeros_like(st_ref)

    a, sh = _affine_from_stats(st_in_ref[...], g_ref[0], b_ref[0], Cin, count)
    h = jnp.maximum(x_ref[0] * a + sh, 0.0)
    zrow = jnp.zeros((1, W + 2, Cin), jnp.float32)
    hp = jnp.concatenate([zrow, h, zrow], axis=0)

    TH = H // strips
    acc_s = jnp.zeros((Cout,), jnp.float32)
    acc_ss = jnp.zeros((Cout,), jnp.float32)
    for si in range(strips):
        r0 = si * TH
        xs = hp[r0:r0 + TH + 2]
        taps = [xs[dy:dy + TH, dx:dx + W, :]
                for dy in range(3) for dx in range(3)]
        patches = jnp.concatenate(taps, axis=-1).reshape(TH * W, 9 * Cin)
        y = jnp.dot(patches, w_ref[...], preferred_element_type=jnp.float32)
        y = y + cb_ref[...]
        acc_s = acc_s + jnp.sum(y, axis=0)
        acc_ss = acc_ss + jnp.sum(y * y, axis=0)
        out_ref[0, r0 // 2:(r0 + TH) // 2] = _pool_border(y, TH, W, Cout)
    st_ref[...] = st_ref[...] + _pack_stats(acc_s, acc_ss, Cout)


def _head_kernel(x_ref, st_in_ref, g_ref, b_ref, wr_ref, rb_ref,
                 wh_ref, hb_ref, out_ref, anch_ref, *, count):
    a, sh = _affine_from_stats(st_in_ref[...], g_ref[0], b_ref[0], 128, count)
    h = jnp.maximum(x_ref[0] * a + sh, 0.0)
    zrow = jnp.zeros((1, 30, 128), jnp.float32)
    hp = jnp.concatenate([zrow, h, zrow], axis=0)
    taps = [hp[dy:dy + 28, dx:dx + 28, :]
            for dy in range(3) for dx in range(3)]
    patches = jnp.concatenate(taps, axis=-1).reshape(784, 1152)
    r = jnp.dot(patches, wr_ref[...], preferred_element_type=jnp.float32)
    r = jnp.maximum(r + rb_ref[...], 0.0)
    out_ref[0] = jnp.dot(r, wh_ref[...],
                         preferred_element_type=jnp.float32) + hb_ref[...]

    ri = jax.lax.broadcasted_iota(jnp.int32, (3136, 4), 0)
    col = jax.lax.broadcasted_iota(jnp.int32, (3136, 4), 1)
    pix = ri // 4
    k = ri % 4
    cx = (jnp.astype(pix % 28, jnp.float32) + 0.5) / 28.0
    cy = (jnp.astype(pix // 28, jnp.float32) + 0.5) / 28.0
    sz = jnp.exp2(jnp.astype(k, jnp.float32)) * (16.0 / 224.0)
    anch_ref[0] = jnp.where(col == 0, cx, jnp.where(col == 1, cy, sz))


def _cparams(n=1):
    return pltpu.CompilerParams(dimension_semantics=("arbitrary",) * n)


def kernel(x, params):
    p = params
    B = x.shape[0]
    f32 = jnp.float32

    xn = jnp.pad(jnp.transpose(x, (0, 2, 3, 1)),
                 ((0, 0), (1, 1), (1, 1), (0, 0)))
    xs = jnp.stack([xn[:, 28 * s:28 * s + 30] for s in range(8)], axis=1)

    def cw9(w):
        return jnp.transpose(w, (2, 3, 1, 0)).reshape(-1, w.shape[0])

    w1, w2, w3, wr = cw9(p['c1w']), cw9(p['c2w']), cw9(p['c3w']), cw9(p['rw'])
    wh = jnp.concatenate([p['cw'].reshape(8, 256),
                          p['ww'].reshape(16, 256)], axis=0).T
    r2 = lambda v: v.reshape(1, -1)
    hb = jnp.concatenate([p['cb'], p['wb']]).reshape(1, 24)

    stspec = pl.BlockSpec((8, 128), lambda *_: (0, 0))
    full = lambda a: pl.BlockSpec(a.shape, lambda *_: (0,) * a.ndim)
    img = lambda s: pl.BlockSpec((1,) + s, lambda b: (b, 0, 0, 0))
    stshape = jax.ShapeDtypeStruct((8, 128), f32)

    p1, st1 = pl.pallas_call(
        _stage1_kernel,
        grid=(B, 8),
        in_specs=[pl.BlockSpec((1, 1, 30, 226, 4),
                               lambda b, s: (b, s, 0, 0, 0)),
                  full(w1), pl.BlockSpec((1, 32), lambda *_: (0, 0))],
        out_specs=[pl.BlockSpec((1, 14, 114, 32), lambda b, s: (b, s, 0, 0)),
                   stspec],
        out_shape=[jax.ShapeDtypeStruct((B, 112, 114, 32), f32), stshape],
        compiler_params=_cparams(2),
    )(xs, w1, r2(p['c1b']))

    p2, st2 = pl.pallas_call(
        functools.partial(_stageN_kernel, H=112, W=112, Cin=32, Cout=64,
                          strips=4, count=float(B * 224 * 224)),
        grid=(B,),
        in_specs=[img((112, 114, 32)), stspec,
                  pl.BlockSpec((1, 32), lambda b: (0, 0)),
                  pl.BlockSpec((1, 32), lambda b: (0, 0)),
                  full(w2), pl.BlockSpec((1, 64), lambda b: (0, 0))],
        out_specs=[img((56, 58, 64)), stspec],
        out_shape=[jax.ShapeDtypeStruct((B, 56, 58, 64), f32), stshape],
        compiler_params=_cparams(),
    )(p1, st1, r2(p['g1']), r2(p['b1']), w2, r2(p['c2b']))

    p3, st3 = pl.pallas_call(
        functools.partial(_stageN_kernel, H=56, W=56, Cin=64, Cout=128,
                          strips=2, count=float(B * 112 * 112)),
        grid=(B,),
        in_specs=[img((56, 58, 64)), stspec,
                  pl.BlockSpec((1, 64), lambda b: (0, 0)),
                  pl.BlockSpec((1, 64), lambda b: (0, 0)),
                  full(w3), pl.BlockSpec((1, 128), lambda b: (0, 0))],
        out_specs=[img((28, 30, 128)), stspec],
        out_shape=[jax.ShapeDtypeStruct((B, 28, 30, 128), f32), stshape],
        compiler_params=_cparams(),
    )(p2, st2, r2(p['g2']), r2(p['b2']), w3, r2(p['c3b']))

    heads, anchors = pl.pallas_call(
        functools.partial(_head_kernel, count=float(B * 56 * 56)),
        grid=(B,),
        in_specs=[img((28, 30, 128)), stspec,
                  pl.BlockSpec((1, 128), lambda b: (0, 0)),
                  pl.BlockSpec((1, 128), lambda b: (0, 0)),
                  full(wr), pl.BlockSpec((1, 256), lambda b: (0, 0)),
                  full(wh), pl.BlockSpec((1, 24), lambda b: (0, 0))],
        out_specs=[pl.BlockSpec((1, 784, 24), lambda b: (b, 0, 0)),
                   pl.BlockSpec((1, 3136, 4), lambda b: (b, 0, 0))],
        out_shape=[jax.ShapeDtypeStruct((B, 784, 24), f32),
                   jax.ShapeDtypeStruct((B, 3136, 4), f32)],
        compiler_params=_cparams(),
    )(p3, st3, r2(p['g3']), r2(p['b3']), wr, r2(p['rb']), wh, hb)

    cls = heads[:, :, :8].reshape(B, 3136, 2)
    reg = heads[:, :, 8:24].reshape(B, 3136, 4)
    return cls, reg, anchors


# speedup vs baseline: 1.4990x; 1.4990x over previous
import functools

import jax
import jax.numpy as jnp
from jax.experimental import pallas as pl
from jax.experimental.pallas import tpu as pltpu

_EPS = 1e-5
_NEG = -1e30


def _affine_from_stats(st, g, b, cin, count):
    s = st[0, :cin]
    ss = st[1, :cin]
    mean = s / count
    var = ss / count - mean * mean
    a = g * jax.lax.rsqrt(var + _EPS)
    sh = b - mean * a
    return a, sh


def _pack_stats(acc_s, acc_ss, cout):
    def row(v):
        v = v.reshape(1, -1)
        if cout < 128:
            v = jnp.concatenate(
                [v, jnp.zeros((1, 128 - cout), jnp.float32)], axis=1)
        return v
    return jnp.concatenate(
        [row(acc_s), row(acc_ss), jnp.zeros((6, 128), jnp.float32)], axis=0)


def _pool_border(y, TH, W, cout):
    yp = jnp.max(y.reshape(TH // 2, 2, W // 2, 2, cout), axis=(1, 3))
    neg = jnp.full((TH // 2, 1, cout), _NEG, jnp.float32)
    return jnp.concatenate([neg, yp, neg], axis=1)


def _stage1_kernel(xf_ref, w_ref, cb_ref, m_ref, out_ref, st_ref):
    @pl.when(pl.program_id(0) == 0)
    def _():
        st_ref[...] = jnp.zeros_like(st_ref)

    xf = xf_ref[0]
    mask = m_ref[...]
    acc_s = jnp.zeros((1, 32), jnp.float32)
    acc_ss = jnp.zeros((1, 32), jnp.float32)
    dn = (((0,), (0,)), ((), ()))
    for si in range(4):
        base = si * 14336
        y = jnp.zeros((14336, 32), jnp.float32)
        for t, (dy, dx) in enumerate((dy, dx) for dy in range(3)
                                     for dx in range(3)):
            off = base + dy * 256 + dx
            y = y + jax.lax.dot_general(
                xf[:, off:off + 14336], w_ref[t * 4:(t + 1) * 4, :], dn,
                preferred_element_type=jnp.float32)
        y = y + cb_ref[...]
        acc_s = acc_s + jnp.dot(mask, y, preferred_element_type=jnp.float32)
        acc_ss = acc_ss + jnp.dot(mask, y * y,
                                  preferred_element_type=jnp.float32)
        v = jnp.max(y.reshape(28, 2, 256, 32), axis=1)
        hm = jnp.max(v.reshape(28, 128, 2, 32), axis=2)[:, :112, :]
        neg = jnp.full((28, 1, 32), _NEG, jnp.float32)
        out_ref[0, si * 28:(si + 1) * 28] = jnp.concatenate(
            [neg, hm, neg], axis=1)
    st_ref[...] = st_ref[...] + _pack_stats(acc_s.reshape(32),
                                            acc_ss.reshape(32), 32)


def _stageN_kernel(x_ref, st_in_ref, g_ref, b_ref, w_ref, cb_ref,
                   out_ref, st_ref, *, H, W, Cin, Cout, strips, count):
    @pl.when(pl.program_id(0) == 0)
    def _():
        st_ref[...] = jnp.zeros_like(st_ref)

    a, sh = _affine_from_stats(st_in_ref[...], g_ref[0], b_ref[0], Cin, count)
    h = jnp.maximum(x_ref[0] * a + sh, 0.0)
    zrow = jnp.zeros((1, W + 2, Cin), jnp.float32)
    hp = jnp.concatenate([zrow, h, zrow], axis=0)

    TH = H // strips
    acc_s = jnp.zeros((Cout,), jnp.float32)
    acc_ss = jnp.zeros((Cout,), jnp.float32)
    for si in range(strips):
        r0 = si * TH
        xs = hp[r0:r0 + TH + 2]
        taps = [xs[dy:dy + TH, dx:dx + W, :]
                for dy in range(3) for dx in range(3)]
        patches = jnp.concatenate(taps, axis=-1).reshape(TH * W, 9 * Cin)
        y = jnp.dot(patches, w_ref[...], preferred_element_type=jnp.float32)
        y = y + cb_ref[...]
        acc_s = acc_s + jnp.sum(y, axis=0)
        acc_ss = acc_ss + jnp.sum(y * y, axis=0)
        out_ref[0, r0 // 2:(r0 + TH) // 2] = _pool_border(y, TH, W, Cout)
    st_ref[...] = st_ref[...] + _pack_stats(acc_s, acc_ss, Cout)


def _head_kernel(x_ref, st_in_ref, g_ref, b_ref, wr_ref, rb_ref,
                 wh_ref, hb_ref, out_ref, anch_ref, *, count):
    a, sh = _affine_from_stats(st_in_ref[...], g_ref[0], b_ref[0], 128, count)
    h = jnp.maximum(x_ref[0] * a + sh, 0.0)
    zrow = jnp.zeros((1, 30, 128), jnp.float32)
    hp = jnp.concatenate([zrow, h, zrow], axis=0)
    taps = [hp[dy:dy + 28, dx:dx + 28, :]
            for dy in range(3) for dx in range(3)]
    patches = jnp.concatenate(taps, axis=-1).reshape(784, 1152)
    r = jnp.dot(patches, wr_ref[...], preferred_element_type=jnp.float32)
    r = jnp.maximum(r + rb_ref[...], 0.0)
    out_ref[0] = jnp.dot(r, wh_ref[...],
                         preferred_element_type=jnp.float32) + hb_ref[...]

    ri = jax.lax.broadcasted_iota(jnp.int32, (3136, 4), 0)
    col = jax.lax.broadcasted_iota(jnp.int32, (3136, 4), 1)
    pix = ri // 4
    k = ri % 4
    cx = (jnp.astype(pix % 28, jnp.float32) + 0.5) / 28.0
    cy = (jnp.astype(pix // 28, jnp.float32) + 0.5) / 28.0
    sz = jnp.exp2(jnp.astype(k, jnp.float32)) * (16.0 / 224.0)
    anch_ref[0] = jnp.where(col == 0, cx, jnp.where(col == 1, cy, sz))


def _cparams(n=1):
    return pltpu.CompilerParams(dimension_semantics=("arbitrary",) * n)


def kernel(x, params):
    p = params
    B = x.shape[0]
    f32 = jnp.float32

    xf = jnp.pad(x, ((0, 0), (0, 0), (1, 2), (1, 31))).reshape(B, 4, 58112)
    mask1 = (jnp.arange(14336, dtype=jnp.int32) % 256 < 224
             ).astype(jnp.float32).reshape(1, 14336)

    def cw9(w):
        return jnp.transpose(w, (2, 3, 1, 0)).reshape(-1, w.shape[0])

    w1, w2, w3, wr = cw9(p['c1w']), cw9(p['c2w']), cw9(p['c3w']), cw9(p['rw'])
    wh = jnp.concatenate([p['cw'].reshape(8, 256),
                          p['ww'].reshape(16, 256)], axis=0).T
    r2 = lambda v: v.reshape(1, -1)
    hb = jnp.concatenate([p['cb'], p['wb']]).reshape(1, 24)

    stspec = pl.BlockSpec((8, 128), lambda *_: (0, 0))
    full = lambda a: pl.BlockSpec(a.shape, lambda *_: (0,) * a.ndim)
    img = lambda s: pl.BlockSpec((1,) + s, lambda b: (b, 0, 0, 0))
    stshape = jax.ShapeDtypeStruct((8, 128), f32)

    p1, st1 = pl.pallas_call(
        _stage1_kernel,
        grid=(B,),
        in_specs=[pl.BlockSpec((1, 4, 58112), lambda b: (b, 0, 0)),
                  full(w1), pl.BlockSpec((1, 32), lambda *_: (0, 0)),
                  pl.BlockSpec((1, 14336), lambda *_: (0, 0))],
        out_specs=[img((112, 114, 32)), stspec],
        out_shape=[jax.ShapeDtypeStruct((B, 112, 114, 32), f32), stshape],
        compiler_params=_cparams(1),
    )(xf, w1, r2(p['c1b']), mask1)

    p2, st2 = pl.pallas_call(
        functools.partial(_stageN_kernel, H=112, W=112, Cin=32, Cout=64,
                          strips=4, count=float(B * 224 * 224)),
        grid=(B,),
        in_specs=[img((112, 114, 32)), stspec,
                  pl.BlockSpec((1, 32), lambda b: (0, 0)),
                  pl.BlockSpec((1, 32), lambda b: (0, 0)),
                  full(w2), pl.BlockSpec((1, 64), lambda b: (0, 0))],
        out_specs=[img((56, 58, 64)), stspec],
        out_shape=[jax.ShapeDtypeStruct((B, 56, 58, 64), f32), stshape],
        compiler_params=_cparams(),
    )(p1, st1, r2(p['g1']), r2(p['b1']), w2, r2(p['c2b']))

    p3, st3 = pl.pallas_call(
        functools.partial(_stageN_kernel, H=56, W=56, Cin=64, Cout=128,
                          strips=2, count=float(B * 112 * 112)),
        grid=(B,),
        in_specs=[img((56, 58, 64)), stspec,
                  pl.BlockSpec((1, 64), lambda b: (0, 0)),
                  pl.BlockSpec((1, 64), lambda b: (0, 0)),
                  full(w3), pl.BlockSpec((1, 128), lambda b: (0, 0))],
        out_specs=[img((28, 30, 128)), stspec],
        out_shape=[jax.ShapeDtypeStruct((B, 28, 30, 128), f32), stshape],
        compiler_params=_cparams(),
    )(p2, st2, r2(p['g2']), r2(p['b2']), w3, r2(p['c3b']))

    heads, anchors = pl.pallas_call(
        functools.partial(_head_kernel, count=float(B * 56 * 56)),
        grid=(B,),
        in_specs=[img((28, 30, 128)), stspec,
                  pl.BlockSpec((1, 128), lambda b: (0, 0)),
                  pl.BlockSpec((1, 128), lambda b: (0, 0)),
                  full(wr), pl.BlockSpec((1, 256), lambda b: (0, 0)),
                  full(wh), pl.BlockSpec((1, 24), lambda b: (0, 0))],
        out_specs=[pl.BlockSpec((1, 784, 24), lambda b: (b, 0, 0)),
                   pl.BlockSpec((1, 3136, 4), lambda b: (b, 0, 0))],
        out_shape=[jax.ShapeDtypeStruct((B, 784, 24), f32),
                   jax.ShapeDtypeStruct((B, 3136, 4), f32)],
        compiler_params=_cparams(),
    )(p3, st3, r2(p['g3']), r2(p['b3']), wr, r2(p['rb']), wh, hb)

    cls = heads[:, :, :8].reshape(B, 3136, 2)
    reg = heads[:, :, 8:24].reshape(B, 3136, 4)
    return cls, reg, anchors


# speedup vs baseline: 1.5002x; 1.0008x over previous
import functools

import jax
import jax.numpy as jnp
from jax.experimental import pallas as pl
from jax.experimental.pallas import tpu as pltpu

_EPS = 1e-5
_NEG = -1e30


def _affine_from_stats(st, g, b, cin, count):
    s = st[0, :cin]
    ss = st[1, :cin]
    mean = s / count
    var = ss / count - mean * mean
    a = g * jax.lax.rsqrt(var + _EPS)
    sh = b - mean * a
    return a, sh


def _pack_stats(acc_s, acc_ss, cout):
    def row(v):
        v = v.reshape(1, -1)
        if cout < 128:
            v = jnp.concatenate(
                [v, jnp.zeros((1, 128 - cout), jnp.float32)], axis=1)
        return v
    return jnp.concatenate(
        [row(acc_s), row(acc_ss), jnp.zeros((6, 128), jnp.float32)], axis=0)


def _pool_border(y, TH, W, cout):
    yp = jnp.max(y.reshape(TH // 2, 2, W // 2, 2, cout), axis=(1, 3))
    neg = jnp.full((TH // 2, 1, cout), _NEG, jnp.float32)
    return jnp.concatenate([neg, yp, neg], axis=1)


def _stage1_kernel(xf_ref, w_ref, cb_ref, m_ref, out_ref, st_ref):
    @pl.when(pl.program_id(0) == 0)
    def _():
        st_ref[...] = jnp.zeros_like(st_ref)

    xf = xf_ref[0]
    mask = m_ref[...]
    acc_s = jnp.zeros((1, 32), jnp.float32)
    acc_ss = jnp.zeros((1, 32), jnp.float32)
    dn = (((0,), (0,)), ((), ()))
    for si in range(4):
        base = si * 14336
        y = jnp.zeros((14336, 32), jnp.float32)
        for t, (dy, dx) in enumerate((dy, dx) for dy in range(3)
                                     for dx in range(3)):
            off = base + dy * 256 + dx
            y = y + jax.lax.dot_general(
                xf[:, off:off + 14336], w_ref[t * 4:(t + 1) * 4, :], dn,
                preferred_element_type=jnp.float32)
        y = y + cb_ref[...]
        acc_s = acc_s + jnp.dot(mask, y, preferred_element_type=jnp.float32)
        acc_ss = acc_ss + jnp.dot(mask, y * y,
                                  preferred_element_type=jnp.float32)
        v = jnp.max(y.reshape(28, 2, 256, 32), axis=1)
        hm = jnp.max(v.reshape(28, 128, 2, 32), axis=2)[:, :112, :]
        neg = jnp.full((28, 1, 32), _NEG, jnp.float32)
        out_ref[0, si * 28:(si + 1) * 28] = jnp.concatenate(
            [neg, hm, neg], axis=1)
    st_ref[...] = st_ref[...] + _pack_stats(acc_s.reshape(32),
                                            acc_ss.reshape(32), 32)


def _stageN_kernel(x_ref, st_in_ref, g_ref, b_ref, w_ref, cb_ref,
                   out_ref, st_ref, *, H, W, Cin, Cout, strips, count):
    @pl.when(pl.program_id(0) == 0)
    def _():
        st_ref[...] = jnp.zeros_like(st_ref)

    a, sh = _affine_from_stats(st_in_ref[...], g_ref[0], b_ref[0], Cin, count)
    h = jnp.maximum(x_ref[0] * a + sh, 0.0)
    zrow = jnp.zeros((1, W + 2, Cin), jnp.float32)
    hp = jnp.concatenate([zrow, h, zrow], axis=0)

    TH = H // strips
    acc_s = jnp.zeros((Cout,), jnp.float32)
    acc_ss = jnp.zeros((Cout,), jnp.float32)
    for si in range(strips):
        r0 = si * TH
        xs = hp[r0:r0 + TH + 2]
        taps = [xs[dy:dy + TH, dx:dx + W, :]
                for dy in range(3) for dx in range(3)]
        patches = jnp.concatenate(taps, axis=-1).reshape(TH * W, 9 * Cin)
        y = jnp.dot(patches, w_ref[...], preferred_element_type=jnp.float32)
        y = y + cb_ref[...]
        acc_s = acc_s + jnp.sum(y, axis=0)
        acc_ss = acc_ss + jnp.sum(y * y, axis=0)
        out_ref[0, r0 // 2:(r0 + TH) // 2] = _pool_border(y, TH, W, Cout)
    st_ref[...] = st_ref[...] + _pack_stats(acc_s, acc_ss, Cout)


def _head_kernel(x_ref, st_in_ref, g_ref, b_ref, wr_ref, rb_ref,
                 wh_ref, hb_ref, out_ref, anch_ref, *, count):
    a, sh = _affine_from_stats(st_in_ref[...], g_ref[0], b_ref[0], 128, count)
    h = jnp.maximum(x_ref[0] * a + sh, 0.0)
    zrow = jnp.zeros((1, 30, 128), jnp.float32)
    hp = jnp.concatenate([zrow, h, zrow], axis=0)
    taps = [hp[dy:dy + 28, dx:dx + 28, :]
            for dy in range(3) for dx in range(3)]
    patches = jnp.concatenate(taps, axis=-1).reshape(784, 1152)
    r = jnp.dot(patches, wr_ref[...], preferred_element_type=jnp.float32)
    r = jnp.maximum(r + rb_ref[...], 0.0)
    out_ref[0] = jnp.dot(r, wh_ref[...],
                         preferred_element_type=jnp.float32) + hb_ref[...]

    ri = jax.lax.broadcasted_iota(jnp.int32, (3136, 4), 0)
    col = jax.lax.broadcasted_iota(jnp.int32, (3136, 4), 1)
    pix = ri // 4
    k = ri % 4
    cx = (jnp.astype(pix % 28, jnp.float32) + 0.5) / 28.0
    cy = (jnp.astype(pix // 28, jnp.float32) + 0.5) / 28.0
    sz = jnp.exp2(jnp.astype(k, jnp.float32)) * (16.0 / 224.0)
    anch_ref[0] = jnp.where(col == 0, cx, jnp.where(col == 1, cy, sz))


def _cparams(n=1, fuse_t=False):
    return pltpu.CompilerParams(dimension_semantics=("arbitrary",) * n,
                                fuse_transposed_lhs_in_matmul=fuse_t)


def kernel(x, params):
    p = params
    B = x.shape[0]
    f32 = jnp.float32

    xf = jnp.pad(x, ((0, 0), (0, 0), (1, 2), (1, 31))).reshape(B, 4, 58112)
    mask1 = (jnp.arange(14336, dtype=jnp.int32) % 256 < 224
             ).astype(jnp.float32).reshape(1, 14336)

    def cw9(w):
        return jnp.transpose(w, (2, 3, 1, 0)).reshape(-1, w.shape[0])

    w1, w2, w3, wr = cw9(p['c1w']), cw9(p['c2w']), cw9(p['c3w']), cw9(p['rw'])
    wh = jnp.concatenate([p['cw'].reshape(8, 256),
                          p['ww'].reshape(16, 256)], axis=0).T
    r2 = lambda v: v.reshape(1, -1)
    hb = jnp.concatenate([p['cb'], p['wb']]).reshape(1, 24)

    stspec = pl.BlockSpec((8, 128), lambda *_: (0, 0))
    full = lambda a: pl.BlockSpec(a.shape, lambda *_: (0,) * a.ndim)
    img = lambda s: pl.BlockSpec((1,) + s, lambda b: (b, 0, 0, 0))
    stshape = jax.ShapeDtypeStruct((8, 128), f32)

    p1, st1 = pl.pallas_call(
        _stage1_kernel,
        grid=(B,),
        in_specs=[pl.BlockSpec((1, 4, 58112), lambda b: (b, 0, 0)),
                  full(w1), pl.BlockSpec((1, 32), lambda *_: (0, 0)),
                  pl.BlockSpec((1, 14336), lambda *_: (0, 0))],
        out_specs=[img((112, 114, 32)), stspec],
        out_shape=[jax.ShapeDtypeStruct((B, 112, 114, 32), f32), stshape],
        compiler_params=_cparams(1, fuse_t=True),
    )(xf, w1, r2(p['c1b']), mask1)

    p2, st2 = pl.pallas_call(
        functools.partial(_stageN_kernel, H=112, W=112, Cin=32, Cout=64,
                          strips=4, count=float(B * 224 * 224)),
        grid=(B,),
        in_specs=[img((112, 114, 32)), stspec,
                  pl.BlockSpec((1, 32), lambda b: (0, 0)),
                  pl.BlockSpec((1, 32), lambda b: (0, 0)),
                  full(w2), pl.BlockSpec((1, 64), lambda b: (0, 0))],
        out_specs=[img((56, 58, 64)), stspec],
        out_shape=[jax.ShapeDtypeStruct((B, 56, 58, 64), f32), stshape],
        compiler_params=_cparams(),
    )(p1, st1, r2(p['g1']), r2(p['b1']), w2, r2(p['c2b']))

    p3, st3 = pl.pallas_call(
        functools.partial(_stageN_kernel, H=56, W=56, Cin=64, Cout=128,
                          strips=2, count=float(B * 112 * 112)),
        grid=(B,),
        in_specs=[img((56, 58, 64)), stspec,
                  pl.BlockSpec((1, 64), lambda b: (0, 0)),
                  pl.BlockSpec((1, 64), lambda b: (0, 0)),
                  full(w3), pl.BlockSpec((1, 128), lambda b: (0, 0))],
        out_specs=[img((28, 30, 128)), stspec],
        out_shape=[jax.ShapeDtypeStruct((B, 28, 30, 128), f32), stshape],
        compiler_params=_cparams(),
    )(p2, st2, r2(p['g2']), r2(p['b2']), w3, r2(p['c3b']))

    heads, anchors = pl.pallas_call(
        functools.partial(_head_kernel, count=float(B * 56 * 56)),
        grid=(B,),
        in_specs=[img((28, 30, 128)), stspec,
                  pl.BlockSpec((1, 128), lambda b: (0, 0)),
                  pl.BlockSpec((1, 128), lambda b: (0, 0)),
                  full(wr), pl.BlockSpec((1, 256), lambda b: (0, 0)),
                  full(wh), pl.BlockSpec((1, 24), lambda b: (0, 0))],
        out_specs=[pl.BlockSpec((1, 784, 24), lambda b: (b, 0, 0)),
                   pl.BlockSpec((1, 3136, 4), lambda b: (b, 0, 0))],
        out_shape=[jax.ShapeDtypeStruct((B, 784, 24), f32),
                   jax.ShapeDtypeStruct((B, 3136, 4), f32)],
        compiler_params=_cparams(),
    )(p3, st3, r2(p['g3']), r2(p['b3']), wr, r2(p['rb']), wh, hb)

    cls = heads[:, :, :8].reshape(B, 3136, 2)
    reg = heads[:, :, 8:24].reshape(B, 3136, 4)
    return cls, reg, anchors


# speedup vs baseline: 3.6784x; 2.4519x over previous
import functools

import jax
import jax.numpy as jnp
from jax.experimental import pallas as pl
from jax.experimental.pallas import tpu as pltpu

_EPS = 1e-5
_NEG = -1e30


def _affine_from_stats(st, g, b, cin, count):
    s = st[0, :cin]
    ss = st[1, :cin]
    mean = s / count
    var = ss / count - mean * mean
    a = g * jax.lax.rsqrt(var + _EPS)
    sh = b - mean * a
    return a, sh


def _pack_stats(acc_s, acc_ss, cout):
    def row(v):
        v = v.reshape(1, -1)
        if cout < 128:
            v = jnp.concatenate(
                [v, jnp.zeros((1, 128 - cout), jnp.float32)], axis=1)
        return v
    return jnp.concatenate(
        [row(acc_s), row(acc_ss), jnp.zeros((6, 128), jnp.float32)], axis=0)


def _pool_border(y, TH, W, cout):
    yp = jnp.max(y.reshape(TH // 2, 2, W // 2, 2, cout), axis=(1, 3))
    neg = jnp.full((TH // 2, 1, cout), _NEG, jnp.float32)
    return jnp.concatenate([neg, yp, neg], axis=1)


def _stage1_kernel(xf_ref, w_ref, cb_ref, m_ref, out_ref, st_ref):
    @pl.when(pl.program_id(0) == 0)
    def _():
        st_ref[...] = jnp.zeros_like(st_ref)

    xf = xf_ref[0]
    mask = m_ref[...]
    acc_s = jnp.zeros((1, 32), jnp.float32)
    acc_ss = jnp.zeros((1, 32), jnp.float32)
    dn = (((0,), (0,)), ((), ()))
    for si in range(4):
        base = si * 14336
        lhs = jnp.concatenate(
            [xf[:, base + dy * 256 + dx:base + dy * 256 + dx + 14336]
             for dy in range(3) for dx in range(3)], axis=0)
        y = jax.lax.dot_general(lhs, w_ref[...], dn,
                                preferred_element_type=jnp.float32)
        y = y + cb_ref[...]
        acc_s = acc_s + jnp.dot(mask, y, preferred_element_type=jnp.float32)
        acc_ss = acc_ss + jnp.dot(mask, y * y,
                                  preferred_element_type=jnp.float32)
        v = jnp.max(y.reshape(28, 2, 256, 32), axis=1)
        hm = jnp.max(v.reshape(28, 128, 2, 32), axis=2)[:, :112, :]
        neg = jnp.full((28, 1, 32), _NEG, jnp.float32)
        out_ref[0, si * 28:(si + 1) * 28] = jnp.concatenate(
            [neg, hm, neg], axis=1)
    st_ref[...] = st_ref[...] + _pack_stats(acc_s.reshape(32),
                                            acc_ss.reshape(32), 32)


def _stageN_kernel(x_ref, st_in_ref, g_ref, b_ref, w_ref, cb_ref,
                   out_ref, st_ref, *, H, W, Cin, Cout, strips, count):
    @pl.when(pl.program_id(0) == 0)
    def _():
        st_ref[...] = jnp.zeros_like(st_ref)

    a, sh = _affine_from_stats(st_in_ref[...], g_ref[0], b_ref[0], Cin, count)
    h = jnp.maximum(x_ref[0] * a + sh, 0.0)
    zrow = jnp.zeros((1, W + 2, Cin), jnp.float32)
    hp = jnp.concatenate([zrow, h, zrow], axis=0)

    TH = H // strips
    acc_s = jnp.zeros((Cout,), jnp.float32)
    acc_ss = jnp.zeros((Cout,), jnp.float32)
    for si in range(strips):
        r0 = si * TH
        xs = hp[r0:r0 + TH + 2]
        taps = [xs[dy:dy + TH, dx:dx + W, :]
                for dy in range(3) for dx in range(3)]
        patches = jnp.concatenate(taps, axis=-1).reshape(TH * W, 9 * Cin)
        y = jnp.dot(patches, w_ref[...], preferred_element_type=jnp.float32)
        y = y + cb_ref[...]
        acc_s = acc_s + jnp.sum(y, axis=0)
        acc_ss = acc_ss + jnp.sum(y * y, axis=0)
        out_ref[0, r0 // 2:(r0 + TH) // 2] = _pool_border(y, TH, W, Cout)
    st_ref[...] = st_ref[...] + _pack_stats(acc_s, acc_ss, Cout)


def _head_kernel(x_ref, st_in_ref, g_ref, b_ref, wr_ref, rb_ref,
                 wh_ref, hb_ref, out_ref, anch_ref, *, count):
    a, sh = _affine_from_stats(st_in_ref[...], g_ref[0], b_ref[0], 128, count)
    h = jnp.maximum(x_ref[0] * a + sh, 0.0)
    zrow = jnp.zeros((1, 30, 128), jnp.float32)
    hp = jnp.concatenate([zrow, h, zrow], axis=0)
    taps = [hp[dy:dy + 28, dx:dx + 28, :]
            for dy in range(3) for dx in range(3)]
    patches = jnp.concatenate(taps, axis=-1).reshape(784, 1152)
    r = jnp.dot(patches, wr_ref[...], preferred_element_type=jnp.float32)
    r = jnp.maximum(r + rb_ref[...], 0.0)
    out_ref[0] = jnp.dot(r, wh_ref[...],
                         preferred_element_type=jnp.float32) + hb_ref[...]

    ri = jax.lax.broadcasted_iota(jnp.int32, (3136, 4), 0)
    col = jax.lax.broadcasted_iota(jnp.int32, (3136, 4), 1)
    pix = ri // 4
    k = ri % 4
    cx = (jnp.astype(pix % 28, jnp.float32) + 0.5) / 28.0
    cy = (jnp.astype(pix // 28, jnp.float32) + 0.5) / 28.0
    sz = jnp.exp2(jnp.astype(k, jnp.float32)) * (16.0 / 224.0)
    anch_ref[0] = jnp.where(col == 0, cx, jnp.where(col == 1, cy, sz))


def _cparams(n=1, fuse_t=False):
    return pltpu.CompilerParams(dimension_semantics=("arbitrary",) * n,
                                fuse_transposed_lhs_in_matmul=fuse_t)


def kernel(x, params):
    p = params
    B = x.shape[0]
    f32 = jnp.float32

    xf = jnp.pad(x, ((0, 0), (0, 0), (1, 2), (1, 31))).reshape(B, 4, 58112)
    mask1 = (jnp.arange(14336, dtype=jnp.int32) % 256 < 224
             ).astype(jnp.float32).reshape(1, 14336)

    def cw9(w):
        return jnp.transpose(w, (2, 3, 1, 0)).reshape(-1, w.shape[0])

    w1, w2, w3, wr = cw9(p['c1w']), cw9(p['c2w']), cw9(p['c3w']), cw9(p['rw'])
    wh = jnp.concatenate([p['cw'].reshape(8, 256),
                          p['ww'].reshape(16, 256)], axis=0).T
    r2 = lambda v: v.reshape(1, -1)
    hb = jnp.concatenate([p['cb'], p['wb']]).reshape(1, 24)

    stspec = pl.BlockSpec((8, 128), lambda *_: (0, 0))
    full = lambda a: pl.BlockSpec(a.shape, lambda *_: (0,) * a.ndim)
    img = lambda s: pl.BlockSpec((1,) + s, lambda b: (b, 0, 0, 0))
    stshape = jax.ShapeDtypeStruct((8, 128), f32)

    p1, st1 = pl.pallas_call(
        _stage1_kernel,
        grid=(B,),
        in_specs=[pl.BlockSpec((1, 4, 58112), lambda b: (b, 0, 0)),
                  full(w1), pl.BlockSpec((1, 32), lambda *_: (0, 0)),
                  pl.BlockSpec((1, 14336), lambda *_: (0, 0))],
        out_specs=[img((112, 114, 32)), stspec],
        out_shape=[jax.ShapeDtypeStruct((B, 112, 114, 32), f32), stshape],
        compiler_params=_cparams(1, fuse_t=True),
    )(xf, w1, r2(p['c1b']), mask1)

    p2, st2 = pl.pallas_call(
        functools.partial(_stageN_kernel, H=112, W=112, Cin=32, Cout=64,
                          strips=4, count=float(B * 224 * 224)),
        grid=(B,),
        in_specs=[img((112, 114, 32)), stspec,
                  pl.BlockSpec((1, 32), lambda b: (0, 0)),
                  pl.BlockSpec((1, 32), lambda b: (0, 0)),
                  full(w2), pl.BlockSpec((1, 64), lambda b: (0, 0))],
        out_specs=[img((56, 58, 64)), stspec],
        out_shape=[jax.ShapeDtypeStruct((B, 56, 58, 64), f32), stshape],
        compiler_params=_cparams(),
    )(p1, st1, r2(p['g1']), r2(p['b1']), w2, r2(p['c2b']))

    p3, st3 = pl.pallas_call(
        functools.partial(_stageN_kernel, H=56, W=56, Cin=64, Cout=128,
                          strips=2, count=float(B * 112 * 112)),
        grid=(B,),
        in_specs=[img((56, 58, 64)), stspec,
                  pl.BlockSpec((1, 64), lambda b: (0, 0)),
                  pl.BlockSpec((1, 64), lambda b: (0, 0)),
                  full(w3), pl.BlockSpec((1, 128), lambda b: (0, 0))],
        out_specs=[img((28, 30, 128)), stspec],
        out_shape=[jax.ShapeDtypeStruct((B, 28, 30, 128), f32), stshape],
        compiler_params=_cparams(),
    )(p2, st2, r2(p['g2']), r2(p['b2']), w3, r2(p['c3b']))

    heads, anchors = pl.pallas_call(
        functools.partial(_head_kernel, count=float(B * 56 * 56)),
        grid=(B,),
        in_specs=[img((28, 30, 128)), stspec,
                  pl.BlockSpec((1, 128), lambda b: (0, 0)),
                  pl.BlockSpec((1, 128), lambda b: (0, 0)),
                  full(wr), pl.BlockSpec((1, 256), lambda b: (0, 0)),
                  full(wh), pl.BlockSpec((1, 24), lambda b: (0, 0))],
        out_specs=[pl.BlockSpec((1, 784, 24), lambda b: (b, 0, 0)),
                   pl.BlockSpec((1, 3136, 4), lambda b: (b, 0, 0))],
        out_shape=[jax.ShapeDtypeStruct((B, 784, 24), f32),
                   jax.ShapeDtypeStruct((B, 3136, 4), f32)],
        compiler_params=_cparams(),
    )(p3, st3, r2(p['g3']), r2(p['b3']), wr, r2(p['rb']), wh, hb)

    cls = heads[:, :, :8].reshape(B, 3136, 2)
    reg = heads[:, :, 8:24].reshape(B, 3136, 4)
    return cls, reg, anchors


# speedup vs baseline: 3.9484x; 1.0734x over previous
import functools

import jax
import jax.numpy as jnp
from jax.experimental import pallas as pl
from jax.experimental.pallas import tpu as pltpu

_EPS = 1e-5
_NEG = -1e30


def _affine_from_stats(st, g, b, cin, count):
    s = st[0, :cin]
    ss = st[1, :cin]
    mean = s / count
    var = ss / count - mean * mean
    a = g * jax.lax.rsqrt(var + _EPS)
    sh = b - mean * a
    return a, sh


def _pack_stats(acc_s, acc_ss, cout):
    def row(v):
        v = v.reshape(1, -1)
        if cout < 128:
            v = jnp.concatenate(
                [v, jnp.zeros((1, 128 - cout), jnp.float32)], axis=1)
        return v
    return jnp.concatenate(
        [row(acc_s), row(acc_ss), jnp.zeros((6, 128), jnp.float32)], axis=0)


def _pool_border(y, TH, W, cout):
    yp = jnp.max(y.reshape(TH // 2, 2, W // 2, 2, cout), axis=(1, 3))
    neg = jnp.full((TH // 2, 1, cout), _NEG, jnp.float32)
    return jnp.concatenate([neg, yp, neg], axis=1)


def _stage1_kernel(xf_ref, w_ref, cb_ref, m_ref, eye_ref, out_ref, st_ref):
    @pl.when(pl.program_id(0) == 0)
    def _():
        st_ref[...] = jnp.zeros_like(st_ref)

    xf = xf_ref[0]
    mask = m_ref[...]
    acc_s = jnp.zeros((1, 32), jnp.float32)
    acc_ss = jnp.zeros((1, 32), jnp.float32)
    dn = (((0,), (0,)), ((), ()))
    for si in range(4):
        base = si * 14336
        lhs = jnp.concatenate(
            [xf[:, base + dy * 256 + dx:base + dy * 256 + dx + 14336]
             for dy in range(3) for dx in range(3)], axis=0)
        yt = jnp.dot(w_ref[...], lhs,
                     preferred_element_type=jnp.float32)
        yt = yt + cb_ref[...]
        ym = yt * mask
        acc_s = acc_s + jnp.sum(ym, axis=1).reshape(1, 32)
        acc_ss = acc_ss + jnp.sum(ym * yt, axis=1).reshape(1, 32)
        y = jax.lax.dot_general(yt, eye_ref[...], dn,
                                preferred_element_type=jnp.float32)
        v = jnp.max(y.reshape(28, 2, 256, 32), axis=1)
        hm = jnp.max(v.reshape(28, 128, 2, 32), axis=2)[:, :112, :]
        neg = jnp.full((28, 1, 32), _NEG, jnp.float32)
        out_ref[0, si * 28:(si + 1) * 28] = jnp.concatenate(
            [neg, hm, neg], axis=1)
    st_ref[...] = st_ref[...] + _pack_stats(acc_s.reshape(32),
                                            acc_ss.reshape(32), 32)


def _stageN_kernel(x_ref, st_in_ref, g_ref, b_ref, w_ref, cb_ref,
                   out_ref, st_ref, *, H, W, Cin, Cout, strips, count):
    @pl.when(pl.program_id(0) == 0)
    def _():
        st_ref[...] = jnp.zeros_like(st_ref)

    a, sh = _affine_from_stats(st_in_ref[...], g_ref[0], b_ref[0], Cin, count)
    h = jnp.maximum(x_ref[0] * a + sh, 0.0)
    zrow = jnp.zeros((1, W + 2, Cin), jnp.float32)
    hp = jnp.concatenate([zrow, h, zrow], axis=0)

    TH = H // strips
    acc_s = jnp.zeros((Cout,), jnp.float32)
    acc_ss = jnp.zeros((Cout,), jnp.float32)
    for si in range(strips):
        r0 = si * TH
        xs = hp[r0:r0 + TH + 2]
        taps = [xs[dy:dy + TH, dx:dx + W, :]
                for dy in range(3) for dx in range(3)]
        patches = jnp.concatenate(taps, axis=-1).reshape(TH * W, 9 * Cin)
        y = jnp.dot(patches, w_ref[...], preferred_element_type=jnp.float32)
        y = y + cb_ref[...]
        acc_s = acc_s + jnp.sum(y, axis=0)
        acc_ss = acc_ss + jnp.sum(y * y, axis=0)
        out_ref[0, r0 // 2:(r0 + TH) // 2] = _pool_border(y, TH, W, Cout)
    st_ref[...] = st_ref[...] + _pack_stats(acc_s, acc_ss, Cout)


def _head_kernel(x_ref, st_in_ref, g_ref, b_ref, wr_ref, rb_ref,
                 wh_ref, hb_ref, out_ref, anch_ref, *, count):
    a, sh = _affine_from_stats(st_in_ref[...], g_ref[0], b_ref[0], 128, count)
    h = jnp.maximum(x_ref[0] * a + sh, 0.0)
    zrow = jnp.zeros((1, 30, 128), jnp.float32)
    hp = jnp.concatenate([zrow, h, zrow], axis=0)
    taps = [hp[dy:dy + 28, dx:dx + 28, :]
            for dy in range(3) for dx in range(3)]
    patches = jnp.concatenate(taps, axis=-1).reshape(784, 1152)
    r = jnp.dot(patches, wr_ref[...], preferred_element_type=jnp.float32)
    r = jnp.maximum(r + rb_ref[...], 0.0)
    out_ref[0] = jnp.dot(r, wh_ref[...],
                         preferred_element_type=jnp.float32) + hb_ref[...]

    ri = jax.lax.broadcasted_iota(jnp.int32, (3136, 4), 0)
    col = jax.lax.broadcasted_iota(jnp.int32, (3136, 4), 1)
    pix = ri // 4
    k = ri % 4
    cx = (jnp.astype(pix % 28, jnp.float32) + 0.5) / 28.0
    cy = (jnp.astype(pix // 28, jnp.float32) + 0.5) / 28.0
    sz = jnp.exp2(jnp.astype(k, jnp.float32)) * (16.0 / 224.0)
    anch_ref[0] = jnp.where(col == 0, cx, jnp.where(col == 1, cy, sz))


def _cparams(n=1, fuse_t=False):
    return pltpu.CompilerParams(dimension_semantics=("arbitrary",) * n,
                                fuse_transposed_lhs_in_matmul=fuse_t)


def kernel(x, params):
    p = params
    B = x.shape[0]
    f32 = jnp.float32

    xf = jnp.pad(x, ((0, 0), (0, 0), (1, 2), (1, 31))).reshape(B, 4, 58112)
    mask1 = (jnp.arange(14336, dtype=jnp.int32) % 256 < 224
             ).astype(jnp.float32).reshape(1, 14336)

    def cw9(w):
        return jnp.transpose(w, (2, 3, 1, 0)).reshape(-1, w.shape[0])

    w1, w2, w3, wr = cw9(p['c1w']), cw9(p['c2w']), cw9(p['c3w']), cw9(p['rw'])
    wh = jnp.concatenate([p['cw'].reshape(8, 256),
                          p['ww'].reshape(16, 256)], axis=0).T
    r2 = lambda v: v.reshape(1, -1)
    hb = jnp.concatenate([p['cb'], p['wb']]).reshape(1, 24)

    stspec = pl.BlockSpec((8, 128), lambda *_: (0, 0))
    full = lambda a: pl.BlockSpec(a.shape, lambda *_: (0,) * a.ndim)
    img = lambda s: pl.BlockSpec((1,) + s, lambda b: (b, 0, 0, 0))
    stshape = jax.ShapeDtypeStruct((8, 128), f32)

    p1, st1 = pl.pallas_call(
        _stage1_kernel,
        grid=(B,),
        in_specs=[pl.BlockSpec((1, 4, 58112), lambda b: (b, 0, 0)),
                  pl.BlockSpec((32, 36), lambda *_: (0, 0)),
                  pl.BlockSpec((32, 1), lambda *_: (0, 0)),
                  pl.BlockSpec((1, 14336), lambda *_: (0, 0)),
                  pl.BlockSpec((32, 32), lambda *_: (0, 0))],
        out_specs=[img((112, 114, 32)), stspec],
        out_shape=[jax.ShapeDtypeStruct((B, 112, 114, 32), f32), stshape],
        compiler_params=_cparams(1, fuse_t=True),
    )(xf, w1.T, p['c1b'].reshape(32, 1), mask1, jnp.eye(32, dtype=f32))

    p2, st2 = pl.pallas_call(
        functools.partial(_stageN_kernel, H=112, W=112, Cin=32, Cout=64,
                          strips=4, count=float(B * 224 * 224)),
        grid=(B,),
        in_specs=[img((112, 114, 32)), stspec,
                  pl.BlockSpec((1, 32), lambda b: (0, 0)),
                  pl.BlockSpec((1, 32), lambda b: (0, 0)),
                  full(w2), pl.BlockSpec((1, 64), lambda b: (0, 0))],
        out_specs=[img((56, 58, 64)), stspec],
        out_shape=[jax.ShapeDtypeStruct((B, 56, 58, 64), f32), stshape],
        compiler_params=_cparams(),
    )(p1, st1, r2(p['g1']), r2(p['b1']), w2, r2(p['c2b']))

    p3, st3 = pl.pallas_call(
        functools.partial(_stageN_kernel, H=56, W=56, Cin=64, Cout=128,
                          strips=2, count=float(B * 112 * 112)),
        grid=(B,),
        in_specs=[img((56, 58, 64)), stspec,
                  pl.BlockSpec((1, 64), lambda b: (0, 0)),
                  pl.BlockSpec((1, 64), lambda b: (0, 0)),
                  full(w3), pl.BlockSpec((1, 128), lambda b: (0, 0))],
        out_specs=[img((28, 30, 128)), stspec],
        out_shape=[jax.ShapeDtypeStruct((B, 28, 30, 128), f32), stshape],
        compiler_params=_cparams(),
    )(p2, st2, r2(p['g2']), r2(p['b2']), w3, r2(p['c3b']))

    heads, anchors = pl.pallas_call(
        functools.partial(_head_kernel, count=float(B * 56 * 56)),
        grid=(B,),
        in_specs=[img((28, 30, 128)), stspec,
                  pl.BlockSpec((1, 128), lambda b: (0, 0)),
                  pl.BlockSpec((1, 128), lambda b: (0, 0)),
                  full(wr), pl.BlockSpec((1, 256), lambda b: (0, 0)),
                  full(wh), pl.BlockSpec((1, 24), lambda b: (0, 0))],
        out_specs=[pl.BlockSpec((1, 784, 24), lambda b: (b, 0, 0)),
                   pl.BlockSpec((1, 3136, 4), lambda b: (b, 0, 0))],
        out_shape=[jax.ShapeDtypeStruct((B, 784, 24), f32),
                   jax.ShapeDtypeStruct((B, 3136, 4), f32)],
        compiler_params=_cparams(),
    )(p3, st3, r2(p['g3']), r2(p['b3']), wr, r2(p['rb']), wh, hb)

    cls = heads[:, :, :8].reshape(B, 3136, 2)
    reg = heads[:, :, 8:24].reshape(B, 3136, 4)
    return cls, reg, anchors
